# baseline (speedup 1.0000x reference)
import functools

import jax
import jax.numpy as jnp
from jax import lax
from jax.experimental import pallas as pl
from jax.experimental.pallas import tpu as pltpu
from jax.experimental.pallas import tpu_sc as plsc

_NC = 2
_NW = 32
_CELLS = 256 * 7 * 7
_RPT = _CELLS // _NW
_NCHUNK = (_RPT + 15) // 16

_mesh = plsc.VectorSubcoreMesh(core_axis_name="c", subcore_axis_name="s")


@functools.partial(
    pl.kernel,
    mesh=_mesh,
    compiler_params=pltpu.CompilerParams(needs_layout_passes=False),
    out_type=[
        jax.ShapeDtypeStruct((_CELLS,), jnp.float32),
        jax.ShapeDtypeStruct((_CELLS * 8,), jnp.float32),
        jax.ShapeDtypeStruct((_CELLS * 20,), jnp.float32),
    ],
    scratch_types=[
        pltpu.VMEM((_RPT * 30,), jnp.float32),
        pltpu.VMEM((_RPT,), jnp.float32),
        pltpu.VMEM((_RPT * 8,), jnp.float32),
        pltpu.VMEM((_RPT * 20,), jnp.float32),
    ],
)
def _sc_head(x_hbm, fg_hbm, loc_hbm, cls_hbm, vin, vfg, vloc, vcls):
    wid = lax.axis_index("s") * _NC + lax.axis_index("c")
    pltpu.sync_copy(x_hbm.at[pl.ds(wid * _RPT * 30, _RPT * 30)], vin)
    iota = lax.iota(jnp.int32, 16)

    def process(start):
        cell = start + iota
        r30 = cell * 30
        ch = [plsc.load_gather(vin, [r30 + c]) for c in range(30)]
        es = [jnp.exp(c) for c in ch[10:30]]
        s = es[0]
        for e in es[1:]:
            s = s + e
        r = 1.0 / s
        r20 = cell * 20
        for c in range(20):
            plsc.store_scatter(vcls, [r20 + c], es[c] * r)
        vfg[pl.ds(start, 16)] = 1.0 / (1.0 + jnp.exp(ch[9] - ch[8]))
        r8 = cell * 8
        for c in range(8):
            plsc.store_scatter(vloc, [r8 + c], 1.0 / (1.0 + jnp.exp(-ch[c])))

    def chunk(j, carry):
        process(j * 16)
        return carry

    lax.fori_loop(0, _RPT // 16, chunk, 0)
    process(_RPT - 16)
    pltpu.sync_copy(vfg, fg_hbm.at[pl.ds(wid * _RPT, _RPT)])
    pltpu.sync_copy(vloc, loc_hbm.at[pl.ds(wid * _RPT * 8, _RPT * 8)])
    pltpu.sync_copy(vcls, cls_hbm.at[pl.ds(wid * _RPT * 20, _RPT * 20)])


def kernel(x):
    xf = x.reshape(_CELLS * 30)
    fg, loc, cls = _sc_head(xf)
    return (fg.reshape(256, 7, 7),
            loc.reshape(256, 7, 7, 2, 4),
            cls.reshape(256, 7, 7, 20))


# speedup vs baseline: 3.2725x; 3.2725x over previous
import functools

import jax
import jax.numpy as jnp
from jax import lax
from jax.experimental import pallas as pl
from jax.experimental.pallas import tpu as pltpu
from jax.experimental.pallas import tpu_sc as plsc

_NC = 2
_NW = 32
_NIJ = 49
_B = 256

_mesh = plsc.VectorSubcoreMesh(core_axis_name="c", subcore_axis_name="s")


@functools.partial(
    pl.kernel,
    mesh=_mesh,
    compiler_params=pltpu.CompilerParams(needs_layout_passes=False),
    out_type=[
        jax.ShapeDtypeStruct((_NIJ * _B,), jnp.float32),
        jax.ShapeDtypeStruct((_NIJ * 8 * _B,), jnp.float32),
        jax.ShapeDtypeStruct((_NIJ * 20 * _B,), jnp.float32),
    ],
    scratch_types=[
        pltpu.VMEM((30 * _B,), jnp.float32),
        pltpu.VMEM((_B,), jnp.float32),
        pltpu.VMEM((8 * _B,), jnp.float32),
        pltpu.VMEM((20 * _B,), jnp.float32),
    ],
)
def _sc_head(x_hbm, fg_hbm, loc_hbm, cls_hbm, vin, vfg, vloc, vcls):
    wid = lax.axis_index("s") * _NC + lax.axis_index("c")

    def do_block(blk):
        pltpu.sync_copy(x_hbm.at[pl.ds(blk * (30 * _B), 30 * _B)], vin)

        def bchunk(bc, carry):
            base = bc * 16
            ch = [vin[pl.ds(c * _B + base, 16)] for c in range(30)]
            es = [jnp.exp(c) for c in ch[10:30]]
            s = es[0]
            for e in es[1:]:
                s = s + e
            r = 1.0 / s
            for c in range(20):
                vcls[pl.ds(c * _B + base, 16)] = es[c] * r
            vfg[pl.ds(base, 16)] = 1.0 / (1.0 + jnp.exp(ch[9] - ch[8]))
            for c in range(8):
                vloc[pl.ds(c * _B + base, 16)] = 1.0 / (1.0 + jnp.exp(-ch[c]))
            return carry

        lax.fori_loop(0, _B // 16, bchunk, 0)
        pltpu.sync_copy(vfg, fg_hbm.at[pl.ds(blk * _B, _B)])
        pltpu.sync_copy(vloc, loc_hbm.at[pl.ds(blk * (8 * _B), 8 * _B)])
        pltpu.sync_copy(vcls, cls_hbm.at[pl.ds(blk * (20 * _B), 20 * _B)])

    do_block(wid)
    extra = wid + _NW

    @pl.when(extra < _NIJ)
    def _():
        do_block(extra)


def kernel(x):
    xf = jnp.transpose(x, (1, 2, 3, 0)).reshape(_NIJ * 30 * _B)
    fgf, locf, clsf = _sc_head(xf)
    fg = jnp.transpose(fgf.reshape(7, 7, _B), (2, 0, 1))
    loc = jnp.transpose(locf.reshape(7, 7, 8, _B), (3, 0, 1, 2)).reshape(
        256, 7, 7, 2, 4)
    cls = jnp.transpose(clsf.reshape(7, 7, 20, _B), (3, 0, 1, 2))
    return (fg, loc, cls)


# speedup vs baseline: 12.5851x; 3.8457x over previous
import jax
import jax.numpy as jnp
from jax.experimental import pallas as pl


def _body(x_ref, fg_ref, loc_ref, cls_ref):
    v = x_ref[...]
    e = jnp.exp(v[:, :, 10:30, :])
    s = jnp.sum(e, axis=2, keepdims=True)
    cls_ref[...] = jnp.transpose(e * (1.0 / s), (0, 2, 1, 3))
    fg_ref[...] = jax.nn.sigmoid(v[:, :, 8, :] - v[:, :, 9, :])
    l = jax.nn.sigmoid(v[0, :, 0:8, :])
    l6 = l.reshape(7, 2, 4, 2, 128)
    loc_ref[...] = jnp.transpose(l6, (0, 1, 3, 2, 4)).reshape(8, 1792)


def kernel(x):
    xt = jnp.transpose(x, (1, 2, 3, 0))
    fgt, loct, clst = pl.pallas_call(
        _body,
        grid=(7,),
        in_specs=[pl.BlockSpec((1, 7, 30, 256), lambda i: (i, 0, 0, 0))],
        out_specs=[
            pl.BlockSpec((1, 7, 256), lambda i: (i, 0, 0)),
            pl.BlockSpec((8, 1792), lambda i: (i, 0)),
            pl.BlockSpec((1, 20, 7, 256), lambda i: (i, 0, 0, 0)),
        ],
        out_shape=[
            jax.ShapeDtypeStruct((7, 7, 256), jnp.float32),
            jax.ShapeDtypeStruct((56, 1792), jnp.float32),
            jax.ShapeDtypeStruct((7, 20, 7, 256), jnp.float32),
        ],
    )(xt)
    fg = jnp.transpose(fgt, (2, 0, 1))
    loc = jnp.transpose(
        loct.reshape(7, 7, 2, 2, 4, 128), (3, 5, 0, 1, 2, 4)
    ).reshape(256, 7, 7, 2, 4)
    cls = jnp.transpose(clst, (3, 0, 2, 1))
    return (fg, loc, cls)


# speedup vs baseline: 13.0187x; 1.0344x over previous
import jax
import jax.numpy as jnp
from jax.experimental import pallas as pl
from jax.experimental.pallas import tpu as pltpu


def _body(x_ref, fg_ref, loc_ref, cls_ref):
    v = x_ref[...]
    e = jnp.exp(v[:, :, 10:30, :])
    s = jnp.sum(e, axis=2, keepdims=True)
    cls_ref[...] = jnp.transpose(e * (1.0 / s), (0, 2, 1, 3))
    fg_ref[...] = jax.nn.sigmoid(v[:, :, 8, :] - v[:, :, 9, :])
    loc_ref[...] = jax.nn.sigmoid(v[:, :, 0:8, :])


def kernel(x):
    xt = jnp.transpose(x, (1, 2, 3, 0))
    xt = pltpu.with_memory_space_constraint(xt, pltpu.MemorySpace.HBM)
    fgt, loct, clst = pl.pallas_call(
        _body,
        grid=(7,),
        in_specs=[pl.BlockSpec((1, 7, 30, 256), lambda i: (i, 0, 0, 0))],
        out_specs=[
            pl.BlockSpec((1, 7, 256), lambda i: (i, 0, 0)),
            pl.BlockSpec((1, 7, 8, 256), lambda i: (i, 0, 0, 0)),
            pl.BlockSpec((1, 20, 7, 256), lambda i: (i, 0, 0, 0)),
        ],
        out_shape=[
            jax.ShapeDtypeStruct((7, 7, 256), jnp.float32),
            jax.ShapeDtypeStruct((7, 7, 8, 256), jnp.float32),
            jax.ShapeDtypeStruct((7, 20, 7, 256), jnp.float32),
        ],
    )(xt)
    fg = jnp.transpose(fgt, (2, 0, 1))
    loc = jnp.transpose(loct, (3, 0, 1, 2)).reshape(256, 7, 7, 2, 4)
    cls = jnp.transpose(clst, (3, 0, 2, 1))
    return (fg, loc, cls)
